# baseline (speedup 1.0000x reference)
import numpy as np

import jax
import jax.numpy as jnp
from jax.experimental import pallas as pl
from jax.experimental.pallas import tpu as pltpu

_E_TILE = 128
_ROWS = 8


def _row_reduce(x, op):
    n = x.shape[-1]
    while n > 1:
        half = n // 2
        x = op(x[:, :half], x[:, half:n])
        n = half
    return x


def _bp_kernel(h_ref, w_ref, hbase_ref, s2_ref, out_ref, mu_s, tot_s, bt_s,
               stage_s):
    i = pl.program_id(0)
    j = pl.program_id(1)
    n_iter = pl.num_programs(0)
    n_tiles = pl.num_programs(1)
    t = _E_TILE
    r_sz = _ROWS
    num_nodes = h_ref.shape[1]

    @pl.when(jnp.logical_and(i == 0, j == 0))
    def _():
        tot_s[0] = jnp.zeros_like(tot_s[0])

    @pl.when(j == 0)
    def _():
        tot_s[(i + 1) % 2] = jnp.zeros_like(tot_s[0])

    @pl.when(i == 0)
    def _():
        mu_s[pl.ds(j * t, t), :] = jnp.zeros((t, num_nodes), jnp.float32)

    bt_s[...] = jnp.broadcast_to(hbase_ref[...] + tot_s[i % 2],
                                 (r_sz, num_nodes))
    pacc = jnp.zeros((r_sz, num_nodes), jnp.float32)

    for r in range(t // r_sz):
        rows_in = pl.ds(r * r_sz, r_sz)
        rows_mu = pl.ds(j * t + r * r_sz, r_sz)
        hm = h_ref[rows_in, :] == 1
        wv = jnp.where(hm, w_ref[rows_in, :], 0.0)
        mn = jnp.where(hm, 0.0, -1e30)
        m = (bt_s[...] - mu_s[rows_mu, :]) - mn
        tt = jnp.tanh(m)
        izf = jnp.where(tt == 0.0, 1.0, 0.0)
        nzv = tt + izf
        p8 = _row_reduce(nzv, jnp.multiply)
        zc8 = _row_reduce(izf, jnp.add)
        selm = (zc8 - izf) == 0.0
        pb = jnp.broadcast_to(p8, (r_sz, num_nodes))
        ratio = (nzv + pb) / (nzv - pb)
        mu_new = jnp.where(selm, s2_ref[rows_in, :] * jnp.log2(ratio), 0.0)
        q_new = mu_new * wv
        stage_s[rows_in, :] = q_new
        pacc = pacc + q_new

    mu_s[pl.ds(j * t, t), :] = stage_s[...]
    tot_s[(i + 1) % 2] += jnp.sum(pacc, axis=0, keepdims=True)

    @pl.when(jnp.logical_and(i == n_iter - 1, j == n_tiles - 1))
    def _():
        out_ref[...] = tot_s[n_iter % 2]


def kernel(l_v, h, s_c, iterations, b, w):
    num_edges, num_nodes = h.shape
    hbase2d = (0.5 * l_v * b).reshape(1, num_nodes)
    s2 = ((1.0 - 2.0 * s_c.astype(jnp.float32))
          * (0.5 * float(np.log(2.0)))).reshape(num_edges, 1)

    t = _E_TILE
    n_tiles = num_edges // t
    tot = pl.pallas_call(
        _bp_kernel,
        grid=(iterations, n_tiles),
        in_specs=[
            pl.BlockSpec((t, num_nodes), lambda i, j: (j, 0)),
            pl.BlockSpec((t, num_nodes), lambda i, j: (j, 0)),
            pl.BlockSpec((1, num_nodes), lambda i, j: (0, 0)),
            pl.BlockSpec((t, 1), lambda i, j: (j, 0)),
        ],
        out_specs=pl.BlockSpec((1, num_nodes), lambda i, j: (0, 0)),
        out_shape=jax.ShapeDtypeStruct((1, num_nodes), jnp.float32),
        scratch_shapes=[
            pltpu.VMEM((num_edges, num_nodes), jnp.float32),
            pltpu.VMEM((2, 1, num_nodes), jnp.float32),
            pltpu.VMEM((_ROWS, num_nodes), jnp.float32),
            pltpu.VMEM((_E_TILE, num_nodes), jnp.float32),
        ],
    )(h, w, hbase2d, s2)

    mu_v = 2.0 * (hbase2d[0] + tot[0])
    return 1.0 / (jnp.exp(mu_v) + 1.0)


# speedup vs baseline: 1.0103x; 1.0103x over previous
import numpy as np

import jax
import jax.numpy as jnp
from jax.experimental import pallas as pl
from jax.experimental.pallas import tpu as pltpu

_E_TILE = 256
_ROWS = 8


def _row_reduce(x, op):
    n = x.shape[-1]
    while n > 1:
        half = n // 2
        x = op(x[:, :half], x[:, half:n])
        n = half
    return x


def _bp_kernel(h_ref, w_ref, hbase_ref, s2_ref, out_ref, mu_s, tot_s, bt_s,
               stage_s):
    i = pl.program_id(0)
    j = pl.program_id(1)
    n_iter = pl.num_programs(0)
    n_tiles = pl.num_programs(1)
    t = _E_TILE
    r_sz = _ROWS
    num_nodes = h_ref.shape[1]

    @pl.when(jnp.logical_and(i == 0, j == 0))
    def _():
        tot_s[0] = jnp.zeros_like(tot_s[0])

    @pl.when(j == 0)
    def _():
        tot_s[(i + 1) % 2] = jnp.zeros_like(tot_s[0])

    @pl.when(i == 0)
    def _():
        mu_s[pl.ds(j * t, t), :] = jnp.zeros((t, num_nodes), jnp.float32)

    bt_s[...] = jnp.broadcast_to(hbase_ref[...] + tot_s[i % 2],
                                 (r_sz, num_nodes))
    pacc = jnp.zeros((r_sz, num_nodes), jnp.float32)

    for r in range(t // r_sz):
        rows_in = pl.ds(r * r_sz, r_sz)
        rows_mu = pl.ds(j * t + r * r_sz, r_sz)
        hf = h_ref[rows_in, :].astype(jnp.float32)
        wv = w_ref[rows_in, :] * hf
        mn = (hf - 1.0) * 1e30
        m = (bt_s[...] - mu_s[rows_mu, :]) - mn
        tt = jnp.tanh(m)
        izf = jnp.where(tt == 0.0, 1.0, 0.0)
        nzv = tt + izf
        p8 = _row_reduce(nzv, jnp.multiply)
        zc8 = _row_reduce(izf, jnp.add)
        selm = (zc8 - izf) == 0.0
        pb = jnp.broadcast_to(p8, (r_sz, num_nodes))
        ratio = (nzv + pb) / (nzv - pb)
        mu_new = jnp.where(selm, s2_ref[rows_in, :] * jnp.log2(ratio), 0.0)
        q_new = mu_new * wv
        stage_s[rows_in, :] = q_new
        pacc = pacc + q_new

    mu_s[pl.ds(j * t, t), :] = stage_s[...]
    tot_s[(i + 1) % 2] += jnp.sum(pacc, axis=0, keepdims=True)

    @pl.when(jnp.logical_and(i == n_iter - 1, j == n_tiles - 1))
    def _():
        out_ref[...] = tot_s[n_iter % 2]


def kernel(l_v, h, s_c, iterations, b, w):
    num_edges, num_nodes = h.shape
    hbase2d = (0.5 * l_v * b).reshape(1, num_nodes)
    s2 = ((1.0 - 2.0 * s_c.astype(jnp.float32))
          * (0.5 * float(np.log(2.0)))).reshape(num_edges, 1)
    h8 = h.astype(jnp.int8)

    t = _E_TILE
    n_tiles = num_edges // t
    tot = pl.pallas_call(
        _bp_kernel,
        grid=(iterations, n_tiles),
        in_specs=[
            pl.BlockSpec((t, num_nodes), lambda i, j: (j, 0)),
            pl.BlockSpec((t, num_nodes), lambda i, j: (j, 0)),
            pl.BlockSpec((1, num_nodes), lambda i, j: (0, 0)),
            pl.BlockSpec((t, 1), lambda i, j: (j, 0)),
        ],
        out_specs=pl.BlockSpec((1, num_nodes), lambda i, j: (0, 0)),
        out_shape=jax.ShapeDtypeStruct((1, num_nodes), jnp.float32),
        scratch_shapes=[
            pltpu.VMEM((num_edges, num_nodes), jnp.float32),
            pltpu.VMEM((2, 1, num_nodes), jnp.float32),
            pltpu.VMEM((_ROWS, num_nodes), jnp.float32),
            pltpu.VMEM((_E_TILE, num_nodes), jnp.float32),
        ],
    )(h8, w, hbase2d, s2)

    mu_v = 2.0 * (hbase2d[0] + tot[0])
    return 1.0 / (jnp.exp(mu_v) + 1.0)
